# Initial kernel scaffold; baseline (speedup 1.0000x reference)
import functools

import jax
import jax.numpy as jnp
import numpy as np
from jax.experimental import pallas as pl
from jax.experimental.pallas import tpu as pltpu

ATOM_DIMS = [119, 5, 12, 12, 10, 6, 6, 2, 2]
CUMD = np.concatenate([[0], np.cumsum(ATOM_DIMS)]).astype(int)
FEAT_SUM = int(np.sum(ATOM_DIMS))
N = 10000
E = 320000
H = 128
BN = 1000
LOGIT_PAD = 256


def _embed_body(feat_ref, emb_ref, out_ref):
    acc = jnp.zeros((BN, H), jnp.float32)
    col_iota = jax.lax.broadcasted_iota(jnp.int32, (BN, 128), 1)
    for i in range(9):
        f = feat_ref[:, i]
        oh = (col_iota == f[:, None]).astype(jnp.float32)
        acc += jnp.dot(oh, emb_ref[i], preferred_element_type=jnp.float32)
    out_ref[...] = acc


def _embed(feat16, emb_pad):
    return pl.pallas_call(
        _embed_body,
        grid=(N // BN,),
        in_specs=[
            pl.BlockSpec((BN, 16), lambda i: (i, 0)),
            pl.BlockSpec((9, 128, H), lambda i: (0, 0, 0)),
        ],
        out_specs=pl.BlockSpec((BN, H), lambda i: (i, 0)),
        out_shape=jax.ShapeDtypeStruct((N, H), jnp.float32),
    )(feat16, emb_pad)


def _dense_body(x_ref, mean_ref, mx_ref, mn_ref, w_ref, b_ref, out_ref):
    wa = w_ref[0:H, :]
    wb = w_ref[H:2 * H, :]
    wc = w_ref[2 * H:3 * H, :]
    acc = jnp.dot(mean_ref[...], wa, preferred_element_type=jnp.float32)
    acc += jnp.dot(mx_ref[...], wb, preferred_element_type=jnp.float32)
    acc += jnp.dot(mn_ref[...], wc, preferred_element_type=jnp.float32)
    out_ref[...] = jnp.maximum(x_ref[...] + acc + b_ref[...], 0.0)


def _dense_update(x, mean, mx, mn, w, b):
    return pl.pallas_call(
        _dense_body,
        grid=(N // BN,),
        in_specs=[
            pl.BlockSpec((BN, H), lambda i: (i, 0)),
            pl.BlockSpec((BN, H), lambda i: (i, 0)),
            pl.BlockSpec((BN, H), lambda i: (i, 0)),
            pl.BlockSpec((BN, H), lambda i: (i, 0)),
            pl.BlockSpec((3 * H, H), lambda i: (0, 0)),
            pl.BlockSpec((1, H), lambda i: (0, 0)),
        ],
        out_specs=pl.BlockSpec((BN, H), lambda i: (i, 0)),
        out_shape=jax.ShapeDtypeStruct((N, H), jnp.float32),
    )(x, mean, mx, mn, w, b.reshape(1, H))


def _readout_body(hx_ref, w_ref, b_ref, tf_ref, mask_ref, nf_ref, loss_ref):
    logits = jnp.dot(hx_ref[...], w_ref[...],
                     preferred_element_type=jnp.float32) + b_ref[...]
    col = jax.lax.broadcasted_iota(jnp.int32, (BN, LOGIT_PAD), 1)
    loss = jnp.zeros((), jnp.float32)
    msk_node = mask_ref[:, 0] != 0
    for i in range(9):
        lo, hi = int(CUMD[i]), int(CUMD[i + 1])
        in_slice = (col >= lo) & (col < hi)
        neg = jnp.where(in_slice, logits, -jnp.inf)
        m = jnp.max(neg, axis=1)
        se = jnp.sum(jnp.where(in_slice, jnp.exp(logits - m[:, None]), 0.0),
                     axis=1)
        logz = m + jnp.log(se)
        tgt = tf_ref[:, i] + lo
        tl = jnp.sum(jnp.where(col == tgt[:, None], logits, 0.0), axis=1)
        loss += jnp.sum(logz - tl)
        amax = jnp.argmax(neg, axis=1).astype(jnp.int32) - lo
        nf_ref[:, i] = jnp.where(msk_node, tf_ref[:, i], amax)
    for i in range(9, 16):
        nf_ref[:, i] = jnp.zeros((BN,), jnp.int32)
    loss_ref[...] = jnp.full((1, 128), loss, jnp.float32)


def _readout(hx, w_pad, b_pad, tf16, mask16):
    return pl.pallas_call(
        _readout_body,
        grid=(N // BN,),
        in_specs=[
            pl.BlockSpec((BN, H), lambda i: (i, 0)),
            pl.BlockSpec((H, LOGIT_PAD), lambda i: (0, 0)),
            pl.BlockSpec((1, LOGIT_PAD), lambda i: (0, 0)),
            pl.BlockSpec((BN, 16), lambda i: (i, 0)),
            pl.BlockSpec((BN, 16), lambda i: (i, 0)),
        ],
        out_specs=[
            pl.BlockSpec((BN, 16), lambda i: (i, 0)),
            pl.BlockSpec((1, 128), lambda i: (i, 0)),
        ],
        out_shape=[
            jax.ShapeDtypeStruct((N, 16), jnp.int32),
            jax.ShapeDtypeStruct((N // BN, 128), jnp.float32),
        ],
    )(hx, w_pad, b_pad, tf16, mask16)


def _final_body(x_ref, mean_ref, mx_ref, mn_ref, w_ref, b_ref, ow_ref,
                ob_ref, out_ref):
    wa = w_ref[0:H, :]
    wb = w_ref[H:2 * H, :]
    wc = w_ref[2 * H:3 * H, :]
    acc = jnp.dot(mean_ref[...], wa, preferred_element_type=jnp.float32)
    acc += jnp.dot(mx_ref[...], wb, preferred_element_type=jnp.float32)
    acc += jnp.dot(mn_ref[...], wc, preferred_element_type=jnp.float32)
    hd = jnp.maximum(x_ref[...] + acc + b_ref[...], 0.0)
    out_ref[...] = jnp.dot(hd, ow_ref[...],
                           preferred_element_type=jnp.float32) + ob_ref[...]


def _final_update(x, mean, mx, mn, w, b, ow_pad, ob_pad):
    return pl.pallas_call(
        _final_body,
        grid=(N // BN,),
        in_specs=[
            pl.BlockSpec((BN, H), lambda i: (i, 0)),
            pl.BlockSpec((BN, H), lambda i: (i, 0)),
            pl.BlockSpec((BN, H), lambda i: (i, 0)),
            pl.BlockSpec((BN, H), lambda i: (i, 0)),
            pl.BlockSpec((3 * H, H), lambda i: (0, 0)),
            pl.BlockSpec((1, H), lambda i: (0, 0)),
            pl.BlockSpec((H, 128), lambda i: (0, 0)),
            pl.BlockSpec((1, 128), lambda i: (0, 0)),
        ],
        out_specs=pl.BlockSpec((BN, 128), lambda i: (i, 0)),
        out_shape=jax.ShapeDtypeStruct((N, 128), jnp.float32),
    )(x, mean, mx, mn, w, b.reshape(1, H), ow_pad, ob_pad)


def _aggregate(x, src, dst, e, invdeg):
    msg = x[src] + e
    sums = jax.ops.segment_sum(msg, dst, N)
    mx = jax.ops.segment_max(msg, dst, N)
    mx = jnp.where(jnp.isfinite(mx), mx, 0.0)
    mn = -jax.ops.segment_max(-msg, dst, N)
    mn = jnp.where(jnp.isfinite(mn), mn, 0.0)
    return sums * invdeg, mx, mn


def _pad9(a):
    return jnp.pad(a.astype(jnp.int32), ((0, 0), (0, 16 - a.shape[1])))


def kernel(feat, true_feat, edge_feat, edge_index, mask,
           gen_node_emb, gen_edge_emb, gen_W1, gen_b1, gen_W2, gen_b2,
           gen_out_W, gen_out_b,
           disc_node_emb, disc_edge_emb, disc_W1, disc_b1, disc_W2, disc_b2,
           disc_out_W, disc_out_b):
    src = edge_index[0]
    dst = edge_index[1]
    feat16 = _pad9(feat)
    tf16 = _pad9(true_feat)
    mask16 = jnp.broadcast_to(mask.astype(jnp.int32), (N, 16))

    combo = (edge_feat[:, 0] + 2 * edge_feat[:, 1]
             + 4 * edge_feat[:, 2]).astype(jnp.int32)
    sel = jnp.array([[b & 1, (b >> 1) & 1, (b >> 2) & 1] for b in range(8)],
                    jnp.float32)
    e8_gen = (gen_edge_emb[0, 0] + gen_edge_emb[1, 0] + gen_edge_emb[2, 0]
              + sel @ (gen_edge_emb[:, 1] - gen_edge_emb[:, 0]))
    e8_disc = (disc_edge_emb[0, 0] + disc_edge_emb[1, 0] + disc_edge_emb[2, 0]
               + sel @ (disc_edge_emb[:, 1] - disc_edge_emb[:, 0]))

    deg = jax.ops.segment_sum(jnp.ones((E,), jnp.float32), dst, N)
    invdeg = (1.0 / jnp.maximum(deg, 1.0))[:, None]

    gne_pad = jnp.pad(gen_node_emb, ((0, 0), (0, 128 - 119), (0, 0)))
    dne_pad = jnp.pad(disc_node_emb, ((0, 0), (0, 128 - 119), (0, 0)))

    x = _embed(feat16, gne_pad)
    e_gen = e8_gen[combo]
    for w, b in ((gen_W1, gen_b1), (gen_W2, gen_b2)):
        mean, mx, mn = _aggregate(x, src, dst, e_gen, invdeg)
        x = _dense_update(x, mean, mx, mn, w, b)

    gow_pad = jnp.pad(gen_out_W, ((0, 0), (0, LOGIT_PAD - FEAT_SUM)))
    gob_pad = jnp.pad(gen_out_b, (0, LOGIT_PAD - FEAT_SUM)).reshape(1, -1)
    nf16, loss_part = _readout(x, gow_pad, gob_pad, tf16, mask16)
    gen_loss = jnp.sum(loss_part[:, 0]) / jnp.float32(N)

    xd = _embed(nf16, dne_pad)
    e_disc = e8_disc[combo]
    mean, mx, mn = _aggregate(xd, src, dst, e_disc, invdeg)
    xd = _dense_update(xd, mean, mx, mn, disc_W1, disc_b1)
    mean, mx, mn = _aggregate(xd, src, dst, e_disc, invdeg)
    dow_pad = jnp.pad(disc_out_W, ((0, 0), (0, 127)))
    dob_pad = jnp.pad(disc_out_b, (0, 127)).reshape(1, -1)
    out = _final_update(xd, mean, mx, mn, disc_W2, disc_b2, dow_pad, dob_pad)
    pred = out[:, 0:1]
    return gen_loss, pred


# speedup vs baseline: 1.0313x; 1.0313x over previous
import functools

import jax
import jax.numpy as jnp
import numpy as np
from jax.experimental import pallas as pl
from jax.experimental.pallas import tpu as pltpu

ATOM_DIMS = [119, 5, 12, 12, 10, 6, 6, 2, 2]
CUMD = np.concatenate([[0], np.cumsum(ATOM_DIMS)]).astype(int)
FEAT_SUM = int(np.sum(ATOM_DIMS))
N = 10000
E = 320000
H = 128
BN = 1000
PREC = jax.lax.Precision.HIGHEST


def _pna_ref(feat_int, edge_index, edge_feat_int, node_emb, edge_emb, Ws, bs):
    n = feat_int.shape[0]
    x = jnp.zeros((n, node_emb.shape[-1]), jnp.float32)
    for i in range(feat_int.shape[1]):
        x = x + node_emb[i][feat_int[:, i]]
    e = jnp.zeros((edge_feat_int.shape[0], edge_emb.shape[-1]), jnp.float32)
    for j in range(edge_feat_int.shape[1]):
        e = e + edge_emb[j][edge_feat_int[:, j]]
    src = edge_index[0]
    dst = edge_index[1]
    deg = jax.ops.segment_sum(jnp.ones((src.shape[0],), jnp.float32), dst, n)
    deg = jnp.maximum(deg, 1.0)[:, None]
    for W, b in zip(Ws, bs):
        msg = x[src] + e
        mean = jax.ops.segment_sum(msg, dst, n) / deg
        mx = jax.ops.segment_max(msg, dst, n)
        mx = jnp.where(jnp.isfinite(mx), mx, 0.0)
        mn = -jax.ops.segment_max(-msg, dst, n)
        mn = jnp.where(jnp.isfinite(mn), mn, 0.0)
        agg = jnp.concatenate([mean, mx, mn], axis=-1)
        x = jax.nn.relu(x + agg @ W + b)
    return x


def _embed_body(feat_ref, emb_ref, out_ref):
    acc = jnp.zeros((BN, H), jnp.float32)
    col_iota = jax.lax.broadcasted_iota(jnp.int32, (BN, 128), 1)
    for i in range(9):
        f = feat_ref[:, i]
        oh = (col_iota == f[:, None]).astype(jnp.float32)
        acc += jnp.dot(oh, emb_ref[i], preferred_element_type=jnp.float32,
                       precision=PREC)
    out_ref[...] = acc


def _embed(feat16, emb_pad):
    return pl.pallas_call(
        _embed_body,
        grid=(N // BN,),
        in_specs=[
            pl.BlockSpec((BN, 16), lambda i: (i, 0)),
            pl.BlockSpec((9, 128, H), lambda i: (0, 0, 0)),
        ],
        out_specs=pl.BlockSpec((BN, H), lambda i: (i, 0)),
        out_shape=jax.ShapeDtypeStruct((N, H), jnp.float32),
    )(feat16, emb_pad)


def _dense_body(x_ref, mean_ref, mx_ref, mn_ref, w_ref, b_ref, out_ref):
    wa = w_ref[0:H, :]
    wb = w_ref[H:2 * H, :]
    wc = w_ref[2 * H:3 * H, :]
    acc = jnp.dot(mean_ref[...], wa, preferred_element_type=jnp.float32,
                  precision=PREC)
    acc += jnp.dot(mx_ref[...], wb, preferred_element_type=jnp.float32,
                   precision=PREC)
    acc += jnp.dot(mn_ref[...], wc, preferred_element_type=jnp.float32,
                   precision=PREC)
    out_ref[...] = jnp.maximum(x_ref[...] + acc + b_ref[...], 0.0)


def _dense_update(x, mean, mx, mn, w, b):
    return pl.pallas_call(
        _dense_body,
        grid=(N // BN,),
        in_specs=[
            pl.BlockSpec((BN, H), lambda i: (i, 0)),
            pl.BlockSpec((BN, H), lambda i: (i, 0)),
            pl.BlockSpec((BN, H), lambda i: (i, 0)),
            pl.BlockSpec((BN, H), lambda i: (i, 0)),
            pl.BlockSpec((3 * H, H), lambda i: (0, 0)),
            pl.BlockSpec((1, H), lambda i: (0, 0)),
        ],
        out_specs=pl.BlockSpec((BN, H), lambda i: (i, 0)),
        out_shape=jax.ShapeDtypeStruct((N, H), jnp.float32),
    )(x, mean, mx, mn, w, b.reshape(1, H))


def _final_body(x_ref, mean_ref, mx_ref, mn_ref, w_ref, b_ref, ow_ref,
                ob_ref, out_ref):
    wa = w_ref[0:H, :]
    wb = w_ref[H:2 * H, :]
    wc = w_ref[2 * H:3 * H, :]
    acc = jnp.dot(mean_ref[...], wa, preferred_element_type=jnp.float32,
                  precision=PREC)
    acc += jnp.dot(mx_ref[...], wb, preferred_element_type=jnp.float32,
                   precision=PREC)
    acc += jnp.dot(mn_ref[...], wc, preferred_element_type=jnp.float32,
                   precision=PREC)
    hd = jnp.maximum(x_ref[...] + acc + b_ref[...], 0.0)
    out_ref[...] = jnp.dot(hd, ow_ref[...], preferred_element_type=jnp.float32,
                           precision=PREC) + ob_ref[...]


def _final_update(x, mean, mx, mn, w, b, ow_pad, ob_pad):
    return pl.pallas_call(
        _final_body,
        grid=(N // BN,),
        in_specs=[
            pl.BlockSpec((BN, H), lambda i: (i, 0)),
            pl.BlockSpec((BN, H), lambda i: (i, 0)),
            pl.BlockSpec((BN, H), lambda i: (i, 0)),
            pl.BlockSpec((BN, H), lambda i: (i, 0)),
            pl.BlockSpec((3 * H, H), lambda i: (0, 0)),
            pl.BlockSpec((1, H), lambda i: (0, 0)),
            pl.BlockSpec((H, 128), lambda i: (0, 0)),
            pl.BlockSpec((1, 128), lambda i: (0, 0)),
        ],
        out_specs=pl.BlockSpec((BN, 128), lambda i: (i, 0)),
        out_shape=jax.ShapeDtypeStruct((N, 128), jnp.float32),
    )(x, mean, mx, mn, w, b.reshape(1, H), ow_pad, ob_pad)


def _aggregate(x, src, dst, e, invdeg):
    msg = x[src] + e
    sums = jax.ops.segment_sum(msg, dst, N)
    mx = jax.ops.segment_max(msg, dst, N)
    mx = jnp.where(jnp.isfinite(mx), mx, 0.0)
    mn = -jax.ops.segment_max(-msg, dst, N)
    mn = jnp.where(jnp.isfinite(mn), mn, 0.0)
    return sums * invdeg, mx, mn


def kernel(feat, true_feat, edge_feat, edge_index, mask,
           gen_node_emb, gen_edge_emb, gen_W1, gen_b1, gen_W2, gen_b2,
           gen_out_W, gen_out_b,
           disc_node_emb, disc_edge_emb, disc_W1, disc_b1, disc_W2, disc_b2,
           disc_out_W, disc_out_b):
    src = edge_index[0]
    dst = edge_index[1]

    hx = _pna_ref(feat, edge_index, edge_feat, gen_node_emb, gen_edge_emb,
                  [gen_W1, gen_W2], [gen_b1, gen_b2])
    logits = hx @ gen_out_W + gen_out_b
    gen_loss = jnp.float32(0.0)
    fakes = []
    for i in range(9):
        sl = logits[:, int(CUMD[i]):int(CUMD[i + 1])]
        tgt = true_feat[:, i]
        logp = jax.nn.log_softmax(sl, axis=1)
        gen_loss = gen_loss - jnp.mean(
            jnp.take_along_axis(logp, tgt[:, None], axis=1))
        probs = jax.nn.softmax(jax.lax.stop_gradient(sl), axis=1)
        fakes.append(jnp.argmax(probs, axis=1))
    fake = jnp.stack(fakes, axis=1)
    new_feat = mask * true_feat + (~mask) * fake

    nf16 = jnp.pad(new_feat.astype(jnp.int32), ((0, 0), (0, 7)))
    dne_pad = jnp.pad(disc_node_emb, ((0, 0), (0, 128 - 119), (0, 0)))
    xd = _embed(nf16, dne_pad)

    combo = (edge_feat[:, 0] + 2 * edge_feat[:, 1]
             + 4 * edge_feat[:, 2]).astype(jnp.int32)
    sel = jnp.array([[b & 1, (b >> 1) & 1, (b >> 2) & 1] for b in range(8)],
                    jnp.float32)
    e8_disc = (disc_edge_emb[0, 0] + disc_edge_emb[1, 0] + disc_edge_emb[2, 0]
               + sel @ (disc_edge_emb[:, 1] - disc_edge_emb[:, 0]))
    e_disc = e8_disc[combo]

    deg = jax.ops.segment_sum(jnp.ones((E,), jnp.float32), dst, N)
    invdeg = (1.0 / jnp.maximum(deg, 1.0))[:, None]

    mean, mx, mn = _aggregate(xd, src, dst, e_disc, invdeg)
    xd = _dense_update(xd, mean, mx, mn, disc_W1, disc_b1)
    mean, mx, mn = _aggregate(xd, src, dst, e_disc, invdeg)
    dow_pad = jnp.pad(disc_out_W, ((0, 0), (0, 127)))
    dob_pad = jnp.pad(disc_out_b, (0, 127)).reshape(1, -1)
    out = _final_update(xd, mean, mx, mn, disc_W2, disc_b2, dow_pad, dob_pad)
    pred = out[:, 0:1]
    return gen_loss, pred
